# Initial kernel scaffold; baseline (speedup 1.0000x reference)
import jax
import jax.numpy as jnp
from jax.experimental import pallas as pl


def kernel(x, dense_w, dense_b, ln_gamma, ln_beta, decoder_w, decoder_b):
    raise NotImplementedError("write your pallas kernel here")


# speedup vs baseline: 2.4408x; 2.4408x over previous
import jax
import jax.numpy as jnp
from jax import lax
from jax.experimental import pallas as pl
from jax.experimental.pallas import tpu as pltpu

_TRANS_B = (((1,), (1,)), ((), ()))
_INV_SQRT2 = 0.7071067811865476
_LN_EPS = 1e-12


def _mlm_head_fused(x_ref, wd_ref, bd_ref, g_ref, be_ref, wv_ref, bv_ref,
                    o_ref, t_ref):
    j = pl.program_id(1)

    @pl.when(j == 0)
    def _transform():
        xb = x_ref[...].astype(jnp.bfloat16)
        wb = wd_ref[...].astype(jnp.bfloat16)
        h = lax.dot_general(xb, wb, _TRANS_B,
                            preferred_element_type=jnp.float32)
        h = h + bd_ref[...]

        h = h * 0.5 * (1.0 + lax.erf(h * _INV_SQRT2))

        inv_h = 1.0 / h.shape[-1]
        u = jnp.sum(h, axis=-1, keepdims=True) * inv_h
        d = h - u
        s = jnp.sum(d * d, axis=-1, keepdims=True) * inv_h
        h = d * lax.rsqrt(s + _LN_EPS)
        h = g_ref[...] * h + be_ref[...]

        t_ref[...] = h.astype(t_ref.dtype)

    logits = lax.dot_general(t_ref[...], wv_ref[...].astype(jnp.bfloat16),
                             _TRANS_B, preferred_element_type=jnp.float32)
    o_ref[...] = (logits + bv_ref[...]).astype(o_ref.dtype)


def _round_up(x, m):
    return (x + m - 1) // m * m


def kernel(x, dense_w, dense_b, ln_gamma, ln_beta, decoder_w, decoder_b,
           *, tv=512, vmem_limit_bytes=64 * 1024 * 1024):
    B, S, H = x.shape
    V = decoder_w.shape[0]
    M = B * S

    x2d = x.reshape(M, H)
    m_pad = _round_up(M, 8)
    if m_pad != M:
        x2d = jnp.pad(x2d, ((0, m_pad - M), (0, 0)))

    tv_eff = min(_round_up(tv, 128), _round_up(V, 128))
    v_pad = _round_up(V, 2 * tv_eff)
    dec_w, dec_b = decoder_w, decoder_b
    if v_pad != V:
        dec_w = jnp.pad(decoder_w, ((0, v_pad - V), (0, 0)))
        dec_b = jnp.pad(decoder_b, ((0, v_pad - V),))

    n_j = v_pad // (2 * tv_eff)

    bd2d = dense_b.reshape(1, H)
    g2d = ln_gamma.reshape(1, H)
    be2d = ln_beta.reshape(1, H)
    bv2d = dec_b.reshape(1, v_pad)

    out = pl.pallas_call(
        _mlm_head_fused,
        out_shape=jax.ShapeDtypeStruct((m_pad, v_pad), x.dtype),
        grid_spec=pltpu.PrefetchScalarGridSpec(
            num_scalar_prefetch=0,
            grid=(2, n_j),
            in_specs=[
                pl.BlockSpec((m_pad, H), lambda i, j: (0, 0)),
                pl.BlockSpec((H, H), lambda i, j: (0, 0)),
                pl.BlockSpec((1, H), lambda i, j: (0, 0)),
                pl.BlockSpec((1, H), lambda i, j: (0, 0)),
                pl.BlockSpec((1, H), lambda i, j: (0, 0)),
                pl.BlockSpec((tv_eff, H),
                             lambda i, j, n=n_j: (i * n + j, 0)),
                pl.BlockSpec((1, tv_eff),
                             lambda i, j, n=n_j: (0, i * n + j)),
            ],
            out_specs=pl.BlockSpec((m_pad, tv_eff),
                                   lambda i, j, n=n_j: (0, i * n + j)),
            scratch_shapes=[pltpu.VMEM((m_pad, H), jnp.bfloat16)],
        ),
        compiler_params=pltpu.CompilerParams(
            dimension_semantics=("parallel", "arbitrary"),
            vmem_limit_bytes=vmem_limit_bytes,
        ),
    )(x2d, dense_w, bd2d, g2d, be2d, dec_w, bv2d)

    return out[:M, :V].reshape(B, S, V)


# speedup vs baseline: 2.9202x; 1.1964x over previous
import jax
import jax.numpy as jnp
from jax import lax
from jax.experimental import pallas as pl
from jax.experimental.pallas import tpu as pltpu

_TRANS_B = (((1,), (1,)), ((), ()))
_INV_SQRT2 = 0.7071067811865476
_LN_EPS = 1e-12


def _mlm_head_fused(x_ref, wd_ref, bd_ref, g_ref, be_ref, wv_ref, bv_ref,
                    o_ref, t_ref):
    j = pl.program_id(1)

    @pl.when(j == 0)
    def _transform():
        xb = x_ref[...].astype(jnp.bfloat16)
        wb = wd_ref[...].astype(jnp.bfloat16)
        h = lax.dot_general(xb, wb, _TRANS_B,
                            preferred_element_type=jnp.float32)
        h = h + bd_ref[...]

        h = h * 0.5 * (1.0 + lax.erf(h * _INV_SQRT2))

        inv_h = 1.0 / h.shape[-1]
        u = jnp.sum(h, axis=-1, keepdims=True) * inv_h
        d = h - u
        s = jnp.sum(d * d, axis=-1, keepdims=True) * inv_h
        h = d * lax.rsqrt(s + _LN_EPS)
        h = g_ref[...] * h + be_ref[...]

        t_ref[...] = h.astype(t_ref.dtype)

    logits = lax.dot_general(t_ref[...], wv_ref[...].astype(jnp.bfloat16),
                             _TRANS_B, preferred_element_type=jnp.float32)
    o_ref[...] = (logits + bv_ref[...]).astype(o_ref.dtype)


def _round_up(x, m):
    return (x + m - 1) // m * m


def kernel(x, dense_w, dense_b, ln_gamma, ln_beta, decoder_w, decoder_b,
           *, tv=1024, vmem_limit_bytes=64 * 1024 * 1024):
    B, S, H = x.shape
    V = decoder_w.shape[0]
    M = B * S

    x2d = x.reshape(M, H)
    m_pad = _round_up(M, 8)
    if m_pad != M:
        x2d = jnp.pad(x2d, ((0, m_pad - M), (0, 0)))

    tv_eff = min(_round_up(tv, 128), _round_up(V, 128))
    v_pad = _round_up(V, 2 * tv_eff)
    dec_w, dec_b = decoder_w, decoder_b
    if v_pad != V:
        dec_w = jnp.pad(decoder_w, ((0, v_pad - V), (0, 0)))
        dec_b = jnp.pad(decoder_b, ((0, v_pad - V),))

    n_j = v_pad // (2 * tv_eff)

    bd2d = dense_b.reshape(1, H)
    g2d = ln_gamma.reshape(1, H)
    be2d = ln_beta.reshape(1, H)
    bv2d = dec_b.reshape(1, v_pad)

    out = pl.pallas_call(
        _mlm_head_fused,
        out_shape=jax.ShapeDtypeStruct((m_pad, v_pad), x.dtype),
        grid_spec=pltpu.PrefetchScalarGridSpec(
            num_scalar_prefetch=0,
            grid=(2, n_j),
            in_specs=[
                pl.BlockSpec((m_pad, H), lambda i, j: (0, 0)),
                pl.BlockSpec((H, H), lambda i, j: (0, 0)),
                pl.BlockSpec((1, H), lambda i, j: (0, 0)),
                pl.BlockSpec((1, H), lambda i, j: (0, 0)),
                pl.BlockSpec((1, H), lambda i, j: (0, 0)),
                pl.BlockSpec((tv_eff, H),
                             lambda i, j, n=n_j: (i * n + j, 0)),
                pl.BlockSpec((1, tv_eff),
                             lambda i, j, n=n_j: (0, i * n + j)),
            ],
            out_specs=pl.BlockSpec((m_pad, tv_eff),
                                   lambda i, j, n=n_j: (0, i * n + j)),
            scratch_shapes=[pltpu.VMEM((m_pad, H), jnp.bfloat16)],
        ),
        compiler_params=pltpu.CompilerParams(
            dimension_semantics=("parallel", "arbitrary"),
            vmem_limit_bytes=vmem_limit_bytes,
        ),
    )(x2d, dense_w, bd2d, g2d, be2d, dec_w, bv2d)

    return out[:M, :V].reshape(B, S, V)


# speedup vs baseline: 3.1080x; 1.0643x over previous
import jax
import jax.numpy as jnp
from jax import lax
from jax.experimental import pallas as pl
from jax.experimental.pallas import tpu as pltpu

_TRANS_B = (((1,), (1,)), ((), ()))
_INV_SQRT2 = 0.7071067811865476
_LN_EPS = 1e-12


def _mlm_head_fused(x_ref, wd_ref, bd_ref, g_ref, be_ref, wv_ref, bv_ref,
                    o_ref, t_ref):
    j = pl.program_id(1)

    @pl.when(j == 0)
    def _transform():
        xb = x_ref[...].astype(jnp.bfloat16)
        wb = wd_ref[...].astype(jnp.bfloat16)
        h = lax.dot_general(xb, wb, _TRANS_B,
                            preferred_element_type=jnp.float32)
        h = h + bd_ref[...]

        h = h * 0.5 * (1.0 + lax.erf(h * _INV_SQRT2))

        inv_h = 1.0 / h.shape[-1]
        u = jnp.sum(h, axis=-1, keepdims=True) * inv_h
        d = h - u
        s = jnp.sum(d * d, axis=-1, keepdims=True) * inv_h
        h = d * lax.rsqrt(s + _LN_EPS)
        h = g_ref[...] * h + be_ref[...]

        t_ref[...] = h.astype(t_ref.dtype)

    logits = lax.dot_general(t_ref[...], wv_ref[...].astype(jnp.bfloat16),
                             _TRANS_B, preferred_element_type=jnp.float32)
    o_ref[...] = (logits + bv_ref[...]).astype(o_ref.dtype)


def _round_up(x, m):
    return (x + m - 1) // m * m


def kernel(x, dense_w, dense_b, ln_gamma, ln_beta, decoder_w, decoder_b,
           *, tv=1536, vmem_limit_bytes=64 * 1024 * 1024):
    B, S, H = x.shape
    V = decoder_w.shape[0]
    M = B * S

    x2d = x.reshape(M, H)
    m_pad = _round_up(M, 8)
    if m_pad != M:
        x2d = jnp.pad(x2d, ((0, m_pad - M), (0, 0)))

    tv_eff = min(_round_up(tv, 128), _round_up(V, 128))
    v_pad = _round_up(V, 2 * tv_eff)
    dec_w, dec_b = decoder_w, decoder_b
    if v_pad != V:
        dec_w = jnp.pad(decoder_w, ((0, v_pad - V), (0, 0)))
        dec_b = jnp.pad(decoder_b, ((0, v_pad - V),))

    n_j = v_pad // (2 * tv_eff)

    bd2d = dense_b.reshape(1, H)
    g2d = ln_gamma.reshape(1, H)
    be2d = ln_beta.reshape(1, H)
    bv2d = dec_b.reshape(1, v_pad)

    out = pl.pallas_call(
        _mlm_head_fused,
        out_shape=jax.ShapeDtypeStruct((m_pad, v_pad), x.dtype),
        grid_spec=pltpu.PrefetchScalarGridSpec(
            num_scalar_prefetch=0,
            grid=(2, n_j),
            in_specs=[
                pl.BlockSpec((m_pad, H), lambda i, j: (0, 0)),
                pl.BlockSpec((H, H), lambda i, j: (0, 0)),
                pl.BlockSpec((1, H), lambda i, j: (0, 0)),
                pl.BlockSpec((1, H), lambda i, j: (0, 0)),
                pl.BlockSpec((1, H), lambda i, j: (0, 0)),
                pl.BlockSpec((tv_eff, H),
                             lambda i, j, n=n_j: (i * n + j, 0)),
                pl.BlockSpec((1, tv_eff),
                             lambda i, j, n=n_j: (0, i * n + j)),
            ],
            out_specs=pl.BlockSpec((m_pad, tv_eff),
                                   lambda i, j, n=n_j: (0, i * n + j)),
            scratch_shapes=[pltpu.VMEM((m_pad, H), jnp.bfloat16)],
        ),
        compiler_params=pltpu.CompilerParams(
            dimension_semantics=("parallel", "arbitrary"),
            vmem_limit_bytes=vmem_limit_bytes,
        ),
    )(x2d, dense_w, bd2d, g2d, be2d, dec_w, bv2d)

    return out[:M, :V].reshape(B, S, V)


# speedup vs baseline: 3.1923x; 1.0271x over previous
import jax
import jax.numpy as jnp
from jax import lax
from jax.experimental import pallas as pl
from jax.experimental.pallas import tpu as pltpu

_TRANS_B = (((1,), (1,)), ((), ()))
_INV_SQRT2 = 0.7071067811865476
_LN_EPS = 1e-12


def _mlm_head_fused(x_ref, wd_ref, bd_ref, g_ref, be_ref, wv_ref, bv_ref,
                    o_ref, t_ref):
    j = pl.program_id(1)

    @pl.when(j == 0)
    def _transform():
        xb = x_ref[...].astype(jnp.bfloat16)
        wb = wd_ref[...].astype(jnp.bfloat16)
        h = lax.dot_general(xb, wb, _TRANS_B,
                            preferred_element_type=jnp.float32)
        h = h + bd_ref[...]

        h = h * 0.5 * (1.0 + lax.erf(h * _INV_SQRT2))

        inv_h = 1.0 / h.shape[-1]
        u = jnp.sum(h, axis=-1, keepdims=True) * inv_h
        d = h - u
        s = jnp.sum(d * d, axis=-1, keepdims=True) * inv_h
        h = d * lax.rsqrt(s + _LN_EPS)
        h = g_ref[...] * h + be_ref[...]

        t_ref[...] = h.astype(t_ref.dtype)

    logits = lax.dot_general(t_ref[...], wv_ref[...].astype(jnp.bfloat16),
                             _TRANS_B, preferred_element_type=jnp.float32)
    o_ref[...] = (logits + bv_ref[...]).astype(o_ref.dtype)


def _round_up(x, m):
    return (x + m - 1) // m * m


def kernel(x, dense_w, dense_b, ln_gamma, ln_beta, decoder_w, decoder_b,
           *, tv=2560, vmem_limit_bytes=100 * 1024 * 1024):
    B, S, H = x.shape
    V = decoder_w.shape[0]
    M = B * S

    x2d = x.reshape(M, H)
    m_pad = _round_up(M, 8)
    if m_pad != M:
        x2d = jnp.pad(x2d, ((0, m_pad - M), (0, 0)))

    tv_eff = min(_round_up(tv, 128), _round_up(V, 128))
    v_pad = _round_up(V, 2 * tv_eff)
    dec_w, dec_b = decoder_w, decoder_b
    if v_pad != V:
        dec_w = jnp.pad(decoder_w, ((0, v_pad - V), (0, 0)))
        dec_b = jnp.pad(decoder_b, ((0, v_pad - V),))

    n_j = v_pad // (2 * tv_eff)

    bd2d = dense_b.reshape(1, H)
    g2d = ln_gamma.reshape(1, H)
    be2d = ln_beta.reshape(1, H)
    bv2d = dec_b.reshape(1, v_pad)

    out = pl.pallas_call(
        _mlm_head_fused,
        out_shape=jax.ShapeDtypeStruct((m_pad, v_pad), x.dtype),
        grid_spec=pltpu.PrefetchScalarGridSpec(
            num_scalar_prefetch=0,
            grid=(2, n_j),
            in_specs=[
                pl.BlockSpec((m_pad, H), lambda i, j: (0, 0)),
                pl.BlockSpec((H, H), lambda i, j: (0, 0)),
                pl.BlockSpec((1, H), lambda i, j: (0, 0)),
                pl.BlockSpec((1, H), lambda i, j: (0, 0)),
                pl.BlockSpec((1, H), lambda i, j: (0, 0)),
                pl.BlockSpec((tv_eff, H),
                             lambda i, j, n=n_j: (i * n + j, 0)),
                pl.BlockSpec((1, tv_eff),
                             lambda i, j, n=n_j: (0, i * n + j)),
            ],
            out_specs=pl.BlockSpec((m_pad, tv_eff),
                                   lambda i, j, n=n_j: (0, i * n + j)),
            scratch_shapes=[pltpu.VMEM((m_pad, H), jnp.bfloat16)],
        ),
        compiler_params=pltpu.CompilerParams(
            dimension_semantics=("parallel", "arbitrary"),
            vmem_limit_bytes=vmem_limit_bytes,
        ),
    )(x2d, dense_w, bd2d, g2d, be2d, dec_w, bv2d)

    return out[:M, :V].reshape(B, S, V)
